# Initial kernel scaffold; baseline (speedup 1.0000x reference)
import jax
import jax.numpy as jnp
from jax.experimental import pallas as pl


def kernel(particle_effect, relation_effect, edges):
    raise NotImplementedError("write your pallas kernel here")


# speedup vs baseline: 4.7736x; 4.7736x over previous
import functools

import jax
import jax.numpy as jnp
from jax import lax
from jax.experimental import pallas as pl
from jax.experimental.pallas import tpu as pltpu
from jax.experimental.pallas import tpu_sc as plsc

_ITRS = 7
_NT = 16
_K = 128
_PAD_ROWS = 8
_RSUB = 32


def _propagate(particle, relation, edg, dst3d, *, N, D, E, C):
  NCH = E // _K
  RPT = N // _NT
  NSUB = RPT // _RSUB
  CSL = D // 16
  NPAIR = C // 2

  mesh = plsc.VectorSubcoreMesh(
      core_axis_name="c", subcore_axis_name="s", num_cores=1)

  @functools.partial(
      pl.kernel,
      out_type=(
          jax.ShapeDtypeStruct((N, D), jnp.float32),
          jax.ShapeDtypeStruct((N, D), jnp.float32),
          jax.ShapeDtypeStruct((N, D), jnp.float32),
      ),
      mesh=mesh,
      scratch_types=[
          pltpu.VMEM((2, 2, 2, _K), jnp.int32),
          pltpu.VMEM((2, _K, D), jnp.float32),
          pltpu.VMEM((1, _K), jnp.int32),
          pltpu.VMEM((_RSUB, D), jnp.float32),
          pltpu.VMEM((_RSUB, D), jnp.float32),
          pltpu.VMEM((_RSUB, D), jnp.float32),
          pltpu.VMEM_SHARED((N + _PAD_ROWS, D), jnp.float32),
          pltpu.SemaphoreType.DMA,
          pltpu.SemaphoreType.DMA,
          pltpu.SemaphoreType.DMA,
      ],
  )
  def k(p_hbm, rel_hbm, edg_hbm, dst3d_hbm,
        cum_hbm, ht_hbm, q_hbm,
        idxb, rowb, dstrow, bufA, bufQ, zbuf, agg, sem0, sem1, semI):
    w = lax.axis_index("s")

    def add_into(dst_ref, src_ref):
      def row(r, carry):
        for l in range(CSL):
          sl = pl.ds(l * 16, 16)
          dst_ref[r, sl] = dst_ref[r, sl] + src_ref[r, sl]
        return carry
      lax.fori_loop(0, _RSUB, row, 0)

    def zrow(r, carry):
      for l in range(CSL):
        zbuf[r, pl.ds(l * 16, 16)] = jnp.zeros((16,), jnp.float32)
      return carry
    lax.fori_loop(0, _RSUB, zrow, 0)

    def zchunk(s_, carry):
      base = w * RPT + s_ * _RSUB
      pltpu.sync_copy(zbuf, agg.at[pl.ds(base, _RSUB)])
      return carry
    lax.fori_loop(0, NSUB, zchunk, 0)

    @pl.when(w == 0)
    def _():
      pltpu.sync_copy(zbuf.at[pl.ds(0, _PAD_ROWS)], agg.at[pl.ds(N, _PAD_ROWS)])

    plsc.subcore_barrier()

    def relchunk(i, carry):
      j = w + i * _NT
      pltpu.sync_copy(rel_hbm.at[pl.ds(j * _K, _K)], rowb.at[0])
      pltpu.sync_copy(dst3d_hbm.at[j], dstrow)
      pltpu.sync_copy(rowb.at[0], agg.at[dstrow.at[0]], add=True)
      return carry
    n_my = (NCH - w + _NT - 1) // _NT
    lax.fori_loop(0, n_my, relchunk, 0)

    plsc.subcore_barrier()

    def initchunk(s_, carry):
      base = w * RPT + s_ * _RSUB
      rows = pl.ds(base, _RSUB)
      pltpu.sync_copy(p_hbm.at[rows], bufQ)
      pltpu.sync_copy(agg.at[rows], bufA)
      pltpu.sync_copy(bufQ, ht_hbm.at[rows])
      add_into(bufQ, bufA)
      pltpu.sync_copy(bufQ, q_hbm.at[rows])
      pltpu.sync_copy(zbuf, cum_hbm.at[rows])
      pltpu.sync_copy(zbuf, agg.at[rows])
      return carry
    lax.fori_loop(0, NSUB, initchunk, 0)

    plsc.subcore_barrier()

    def one_iter(it, carry):
      del it
      pltpu.sync_copy(edg_hbm.at[w, 0], idxb.at[0])
      pltpu.async_copy(ht_hbm.at[idxb.at[0, 0, 0]], rowb.at[0], sem0)
      pltpu.async_copy(edg_hbm.at[w, 1], idxb.at[1], semI)

      def pair(t, c2):
        s_ = t % 2
        pltpu.make_async_copy(
            ht_hbm.at[idxb.at[s_, 0, 0]], rowb.at[0], sem0).wait()
        pltpu.async_copy(ht_hbm.at[idxb.at[s_, 1, 0]], rowb.at[1], sem1)
        pltpu.sync_copy(rowb.at[0], agg.at[idxb.at[s_, 0, 1]], add=True)

        @pl.when(t + 1 < NPAIR)
        def _():
          pltpu.make_async_copy(
              edg_hbm.at[w, t + 1], idxb.at[1 - s_], semI).wait()
          pltpu.async_copy(
              ht_hbm.at[idxb.at[1 - s_, 0, 0]], rowb.at[0], sem0)

        pltpu.make_async_copy(
            ht_hbm.at[idxb.at[s_, 1, 0]], rowb.at[1], sem1).wait()
        pltpu.sync_copy(rowb.at[1], agg.at[idxb.at[s_, 1, 1]], add=True)

        @pl.when(t + 2 < NPAIR)
        def _():
          pltpu.async_copy(edg_hbm.at[w, t + 2], idxb.at[s_], semI)
        return c2
      lax.fori_loop(0, NPAIR, pair, 0)

      plsc.subcore_barrier()

      def drain(s_, carry2):
        base = w * RPT + s_ * _RSUB
        rows = pl.ds(base, _RSUB)
        pltpu.sync_copy(agg.at[rows], bufA)
        pltpu.sync_copy(zbuf, agg.at[rows])
        pltpu.sync_copy(q_hbm.at[rows], bufQ)
        add_into(bufQ, bufA)
        pltpu.sync_copy(bufQ, ht_hbm.at[rows])
        pltpu.sync_copy(cum_hbm.at[rows], bufA)
        add_into(bufA, bufQ)
        pltpu.sync_copy(bufA, cum_hbm.at[rows])
        return carry2
      lax.fori_loop(0, NSUB, drain, 0)

      plsc.subcore_barrier()
      return carry

    lax.fori_loop(0, _ITRS, one_iter, 0)

  return k(particle, relation, edg, dst3d)


def kernel(particle_effect, relation_effect, edges):
  N, D = particle_effect.shape
  E = relation_effect.shape[0]
  assert E % _K == 0 and D % 16 == 0

  src = edges[0].astype(jnp.int32)
  dst = edges[1].astype(jnp.int32)

  NP = -(-N // (_NT * _K)) * (_NT * _K)
  p_pad = jnp.pad(particle_effect, ((0, NP - N), (0, 0))) if NP != N \
      else particle_effect

  C = -(-E // (_NT * _K))
  C = -(-C // 8) * 8
  tot = _NT * C * _K
  pad = tot - E
  if pad:
    ar = jnp.arange(pad, dtype=jnp.int32)
    src_p = jnp.concatenate([src, (ar * 37) % N])
    dst_p = jnp.concatenate([dst, NP + (ar % _PAD_ROWS)])
  else:
    src_p, dst_p = src, dst
  edg = jnp.stack(
      [src_p.reshape(_NT, C, _K), dst_p.reshape(_NT, C, _K)], axis=2
  ).reshape(_NT, C // 2, 2, 2, _K)
  dst3d = dst.reshape(E // _K, 1, _K)

  cum, _ht, _q = _propagate(
      p_pad, relation_effect, edg, dst3d, N=NP, D=D, E=E, C=C)
  return cum[:N]


# speedup vs baseline: 4.7828x; 1.0019x over previous
import functools

import jax
import jax.numpy as jnp
from jax import lax
from jax.experimental import pallas as pl
from jax.experimental.pallas import tpu as pltpu
from jax.experimental.pallas import tpu_sc as plsc

_ITRS = 7
_NT = 16
_K = 128
_PAD_ROWS = 8
_RSUB = 32


def _propagate(particle, relation, edg, dst3d, *, N, D, E, C):
  NCH = E // _K
  RPT = N // _NT
  NSUB = RPT // _RSUB
  CSL = D // 16
  NPAIR = C // 2

  mesh = plsc.VectorSubcoreMesh(
      core_axis_name="c", subcore_axis_name="s", num_cores=1)

  @functools.partial(
      pl.kernel,
      out_type=(
          jax.ShapeDtypeStruct((N, D), jnp.float32),
          jax.ShapeDtypeStruct((N, D), jnp.float32),
          jax.ShapeDtypeStruct((N, D), jnp.float32),
      ),
      mesh=mesh,
      scratch_types=[
          pltpu.VMEM((2, 2, 2, _K), jnp.int32),
          pltpu.VMEM((2, _K, D), jnp.float32),
          pltpu.VMEM((1, _K), jnp.int32),
          pltpu.VMEM((_RSUB, D), jnp.float32),
          pltpu.VMEM((_RSUB, D), jnp.float32),
          pltpu.VMEM((_RSUB, D), jnp.float32),
          pltpu.VMEM_SHARED((N + _PAD_ROWS, D), jnp.float32),
          pltpu.SemaphoreType.DMA,
          pltpu.SemaphoreType.DMA,
          pltpu.SemaphoreType.DMA,
      ],
  )
  def k(p_hbm, rel_hbm, edg_hbm, dst3d_hbm,
        cum_hbm, ht_hbm, q_hbm,
        idxb, rowb, dstrow, bufA, bufQ, zbuf, agg, sem0, sem1, semI):
    w = lax.axis_index("s")

    def add_into(dst_ref, src_ref):
      def row(r, carry):
        for l in range(CSL):
          sl = pl.ds(l * 16, 16)
          dst_ref[r, sl] = dst_ref[r, sl] + src_ref[r, sl]
        return carry
      lax.fori_loop(0, _RSUB, row, 0)

    def zrow(r, carry):
      for l in range(CSL):
        zbuf[r, pl.ds(l * 16, 16)] = jnp.zeros((16,), jnp.float32)
      return carry
    lax.fori_loop(0, _RSUB, zrow, 0)

    def zchunk(s_, carry):
      base = w * RPT + s_ * _RSUB
      pltpu.sync_copy(zbuf, agg.at[pl.ds(base, _RSUB)])
      return carry
    lax.fori_loop(0, NSUB, zchunk, 0)

    @pl.when(w == 0)
    def _():
      pltpu.sync_copy(zbuf.at[pl.ds(0, _PAD_ROWS)], agg.at[pl.ds(N, _PAD_ROWS)])

    plsc.subcore_barrier()

    with jax.named_scope("relpass"):
      def relchunk(i, carry):
        j = w + i * _NT
        pltpu.sync_copy(rel_hbm.at[pl.ds(j * _K, _K)], rowb.at[0])
        pltpu.sync_copy(dst3d_hbm.at[j], dstrow)
        pltpu.sync_copy(rowb.at[0], agg.at[dstrow.at[0]], add=True)
        return carry
      n_my = (NCH - w + _NT - 1) // _NT
      lax.fori_loop(0, n_my, relchunk, 0)

    plsc.subcore_barrier()

    def initchunk(s_, carry):
      base = w * RPT + s_ * _RSUB
      rows = pl.ds(base, _RSUB)
      pltpu.sync_copy(p_hbm.at[rows], bufQ)
      pltpu.sync_copy(agg.at[rows], bufA)
      pltpu.sync_copy(bufQ, ht_hbm.at[rows])
      add_into(bufQ, bufA)
      pltpu.sync_copy(bufQ, q_hbm.at[rows])
      pltpu.sync_copy(zbuf, cum_hbm.at[rows])
      pltpu.sync_copy(zbuf, agg.at[rows])
      return carry
    lax.fori_loop(0, NSUB, initchunk, 0)

    plsc.subcore_barrier()

    def one_iter(it, carry):
      del it
      scope_sc = jax.named_scope("scatter_phase")
      scope_sc.__enter__()
      pltpu.sync_copy(edg_hbm.at[w, 0], idxb.at[0])
      pltpu.async_copy(ht_hbm.at[idxb.at[0, 0, 0]], rowb.at[0], sem0)
      pltpu.async_copy(edg_hbm.at[w, 1], idxb.at[1], semI)

      def pair(t, c2):
        s_ = t % 2
        pltpu.make_async_copy(
            ht_hbm.at[idxb.at[s_, 0, 0]], rowb.at[0], sem0).wait()
        pltpu.async_copy(ht_hbm.at[idxb.at[s_, 1, 0]], rowb.at[1], sem1)
        pltpu.sync_copy(rowb.at[0], agg.at[idxb.at[s_, 0, 1]], add=True)

        @pl.when(t + 1 < NPAIR)
        def _():
          pltpu.make_async_copy(
              edg_hbm.at[w, t + 1], idxb.at[1 - s_], semI).wait()
          pltpu.async_copy(
              ht_hbm.at[idxb.at[1 - s_, 0, 0]], rowb.at[0], sem0)

        pltpu.make_async_copy(
            ht_hbm.at[idxb.at[s_, 1, 0]], rowb.at[1], sem1).wait()
        pltpu.sync_copy(rowb.at[1], agg.at[idxb.at[s_, 1, 1]], add=True)

        @pl.when(t + 2 < NPAIR)
        def _():
          pltpu.async_copy(edg_hbm.at[w, t + 2], idxb.at[s_], semI)
        return c2
      lax.fori_loop(0, NPAIR, pair, 0)
      scope_sc.__exit__(None, None, None)

      plsc.subcore_barrier()

      scope_dr = jax.named_scope("drain_phase")
      scope_dr.__enter__()
      def drain(s_, carry2):
        base = w * RPT + s_ * _RSUB
        rows = pl.ds(base, _RSUB)
        pltpu.sync_copy(agg.at[rows], bufA)
        pltpu.sync_copy(zbuf, agg.at[rows])
        pltpu.sync_copy(q_hbm.at[rows], bufQ)
        add_into(bufQ, bufA)
        pltpu.sync_copy(bufQ, ht_hbm.at[rows])
        pltpu.sync_copy(cum_hbm.at[rows], bufA)
        add_into(bufA, bufQ)
        pltpu.sync_copy(bufA, cum_hbm.at[rows])
        return carry2
      lax.fori_loop(0, NSUB, drain, 0)
      scope_dr.__exit__(None, None, None)

      plsc.subcore_barrier()
      return carry

    lax.fori_loop(0, _ITRS, one_iter, 0)

  return k(particle, relation, edg, dst3d)


def kernel(particle_effect, relation_effect, edges):
  N, D = particle_effect.shape
  E = relation_effect.shape[0]
  assert E % _K == 0 and D % 16 == 0

  src = edges[0].astype(jnp.int32)
  dst = edges[1].astype(jnp.int32)

  NP = -(-N // (_NT * _K)) * (_NT * _K)
  p_pad = jnp.pad(particle_effect, ((0, NP - N), (0, 0))) if NP != N \
      else particle_effect

  C = -(-E // (_NT * _K))
  C = -(-C // 8) * 8
  tot = _NT * C * _K
  pad = tot - E
  if pad:
    ar = jnp.arange(pad, dtype=jnp.int32)
    src_p = jnp.concatenate([src, (ar * 37) % N])
    dst_p = jnp.concatenate([dst, NP + (ar % _PAD_ROWS)])
  else:
    src_p, dst_p = src, dst
  edg = jnp.stack(
      [src_p.reshape(_NT, C, _K), dst_p.reshape(_NT, C, _K)], axis=2
  ).reshape(_NT, C // 2, 2, 2, _K)
  dst3d = dst.reshape(E // _K, 1, _K)

  cum, _ht, _q = _propagate(
      p_pad, relation_effect, edg, dst3d, N=NP, D=D, E=E, C=C)
  return cum[:N]


# speedup vs baseline: 5.0680x; 1.0596x over previous
import functools

import jax
import jax.numpy as jnp
from jax import lax
from jax.experimental import pallas as pl
from jax.experimental.pallas import tpu as pltpu
from jax.experimental.pallas import tpu_sc as plsc

_ITRS = 7
_NC = 2
_NT = 16
_K = 128
_RSUB = 32


def _mesh():
  return plsc.VectorSubcoreMesh(
      core_axis_name="c", subcore_axis_name="s", num_cores=_NC)


def _make_phase0(N, D, E):
  NCH = E // _K
  HN = N // _NC
  RPT = HN // _NT
  NSUB = RPT // _RSUB
  CSL = D // 16

  @functools.partial(
      pl.kernel,
      out_type=(
          jax.ShapeDtypeStruct((N, D), jnp.float32),
          jax.ShapeDtypeStruct((N, D), jnp.float32),
          jax.ShapeDtypeStruct((N, D), jnp.float32),
      ),
      mesh=_mesh(),
      scratch_types=[
          pltpu.VMEM((1, _K, D), jnp.float32),
          pltpu.VMEM((1, _K), jnp.int32),
          pltpu.VMEM((_RSUB, D), jnp.float32),
          pltpu.VMEM((_RSUB, D), jnp.float32),
          pltpu.VMEM((_RSUB, D), jnp.float32),
          pltpu.VMEM_SHARED((HN + 8, D), jnp.float32),
      ],
  )
  def k0(p_hbm, rel_hbm, dst3d_hbm, q_hbm, ht_hbm, cum_hbm,
         rowb, dstrow, bufA, bufQ, zbuf, agg):
    cid = lax.axis_index("c")
    w = lax.axis_index("s")

    def add_into(dst_ref, src_ref):
      def row(r, carry):
        for l in range(CSL):
          sl = pl.ds(l * 16, 16)
          dst_ref[r, sl] = dst_ref[r, sl] + src_ref[r, sl]
        return carry
      lax.fori_loop(0, _RSUB, row, 0)

    def zrow(r, carry):
      for l in range(CSL):
        zbuf[r, pl.ds(l * 16, 16)] = jnp.zeros((16,), jnp.float32)
      return carry
    lax.fori_loop(0, _RSUB, zrow, 0)

    def zchunk(s_, carry):
      pltpu.sync_copy(zbuf, agg.at[pl.ds(w * RPT + s_ * _RSUB, _RSUB)])
      return carry
    lax.fori_loop(0, NSUB, zchunk, 0)

    plsc.subcore_barrier()

    def relchunk(i, carry):
      j = w + i * _NT
      pltpu.sync_copy(rel_hbm.at[pl.ds(j * _K, _K)], rowb.at[0])
      pltpu.sync_copy(dst3d_hbm.at[cid, j], dstrow)
      pltpu.sync_copy(
          rowb.at[0],
          agg.at[plsc.Indices(dstrow.at[0], ignored_value=-1)],
          add=True)
      return carry
    n_my = (NCH - w + _NT - 1) // _NT
    lax.fori_loop(0, n_my, relchunk, 0)

    plsc.subcore_barrier()

    def initchunk(s_, carry):
      rows = pl.ds(w * RPT + s_ * _RSUB, _RSUB)
      grows = pl.ds(cid * HN + w * RPT + s_ * _RSUB, _RSUB)
      pltpu.sync_copy(p_hbm.at[grows], bufQ)
      pltpu.sync_copy(agg.at[rows], bufA)
      pltpu.sync_copy(bufQ, ht_hbm.at[grows])
      add_into(bufQ, bufA)
      pltpu.sync_copy(bufQ, q_hbm.at[grows])
      pltpu.sync_copy(zbuf, cum_hbm.at[grows])
      return carry
    lax.fori_loop(0, NSUB, initchunk, 0)

  return k0


def _make_iter(N, D, C):
  HN = N // _NC
  RPT = HN // _NT
  NSUB = RPT // _RSUB
  CSL = D // 16

  @functools.partial(
      pl.kernel,
      out_type=(
          jax.ShapeDtypeStruct((N, D), jnp.float32),
          jax.ShapeDtypeStruct((N, D), jnp.float32),
      ),
      mesh=_mesh(),
      scratch_types=[
          pltpu.VMEM((C, _K), jnp.int32),
          pltpu.VMEM((C, _K), jnp.int32),
          pltpu.VMEM((2, _K, D), jnp.float32),
          pltpu.VMEM((_RSUB, D), jnp.float32),
          pltpu.VMEM((_RSUB, D), jnp.float32),
          pltpu.VMEM((_RSUB, D), jnp.float32),
          pltpu.VMEM_SHARED((HN + 8, D), jnp.float32),
          pltpu.SemaphoreType.DMA,
          pltpu.SemaphoreType.DMA,
      ],
  )
  def kit(srcp_hbm, dstp_hbm, q_hbm, ht_hbm, cum_hbm,
          htn_hbm, cumn_hbm,
          srcv, dstv, rowb, bufA, bufQ, zbuf, agg, sem0, sem1):
    cid = lax.axis_index("c")
    w = lax.axis_index("s")

    def add_into(dst_ref, src_ref):
      def row(r, carry):
        for l in range(CSL):
          sl = pl.ds(l * 16, 16)
          dst_ref[r, sl] = dst_ref[r, sl] + src_ref[r, sl]
        return carry
      lax.fori_loop(0, _RSUB, row, 0)

    pltpu.sync_copy(srcp_hbm.at[cid, w], srcv)
    pltpu.sync_copy(dstp_hbm.at[cid, w], dstv)

    def zrow(r, carry):
      for l in range(CSL):
        zbuf[r, pl.ds(l * 16, 16)] = jnp.zeros((16,), jnp.float32)
      return carry
    lax.fori_loop(0, _RSUB, zrow, 0)

    def zchunk(s_, carry):
      pltpu.sync_copy(zbuf, agg.at[pl.ds(w * RPT + s_ * _RSUB, _RSUB)])
      return carry
    lax.fori_loop(0, NSUB, zchunk, 0)

    @pl.when(w == 0)
    def _():
      pltpu.sync_copy(zbuf.at[pl.ds(0, 8)], agg.at[pl.ds(HN, 8)])

    plsc.subcore_barrier()

    def gsrc(c):
      return ht_hbm.at[plsc.Indices(srcv.at[c], ignored_value=-1)]

    pltpu.async_copy(gsrc(0), rowb.at[0], sem0)

    def pair(t, c2):
      c0 = 2 * t
      c1 = c0 + 1
      pltpu.make_async_copy(gsrc(c0), rowb.at[0], sem0).wait()
      pltpu.async_copy(gsrc(c1), rowb.at[1], sem1)
      pltpu.sync_copy(
          rowb.at[0],
          agg.at[plsc.Indices(dstv.at[c0], ignored_value=-1)],
          add=True)
      pltpu.make_async_copy(gsrc(c1), rowb.at[1], sem1).wait()
      nxt = c0 + 2
      @pl.when(nxt < C)
      def _():
        pltpu.async_copy(gsrc(nxt), rowb.at[0], sem0)
      pltpu.sync_copy(
          rowb.at[1],
          agg.at[plsc.Indices(dstv.at[c1], ignored_value=-1)],
          add=True)
      return c2
    lax.fori_loop(0, C // 2, pair, 0)

    plsc.subcore_barrier()

    def drain(s_, carry2):
      rows = pl.ds(w * RPT + s_ * _RSUB, _RSUB)
      grows = pl.ds(cid * HN + w * RPT + s_ * _RSUB, _RSUB)
      pltpu.sync_copy(agg.at[rows], bufA)
      pltpu.sync_copy(q_hbm.at[grows], bufQ)
      add_into(bufQ, bufA)
      pltpu.sync_copy(bufQ, htn_hbm.at[grows])
      pltpu.sync_copy(cum_hbm.at[grows], bufA)
      add_into(bufA, bufQ)
      pltpu.sync_copy(bufA, cumn_hbm.at[grows])
      return carry2
    lax.fori_loop(0, NSUB, drain, 0)

  return kit


def kernel(particle_effect, relation_effect, edges):
  N, D = particle_effect.shape
  E = relation_effect.shape[0]
  assert E % _K == 0 and D % 16 == 0

  src = edges[0].astype(jnp.int32)
  dst = edges[1].astype(jnp.int32)

  NP = -(-N // (_NC * _NT * _RSUB)) * (_NC * _NT * _RSUB)
  HN = NP // _NC
  p_pad = jnp.pad(particle_effect, ((0, NP - N), (0, 0))) if NP != N \
      else particle_effect

  C = -(-E // (_NT * _K))
  C = -(-C // 8) * 8
  tot = _NT * C * _K
  pad = tot - E
  if pad:
    neg = -jnp.ones((pad,), jnp.int32)
    src_p = jnp.concatenate([src, neg])
    dst_p = jnp.concatenate([dst, neg])
  else:
    src_p, dst_p = src, dst

  own0 = (dst_p >= 0) & (dst_p < HN)
  own1 = dst_p >= HN
  srcp = jnp.stack([
      jnp.where(own0, src_p, -1).reshape(_NT, C, _K),
      jnp.where(own1, src_p, -1).reshape(_NT, C, _K),
  ])
  dstp = jnp.stack([
      jnp.where(own0, dst_p, -1).reshape(_NT, C, _K),
      jnp.where(own1, dst_p - HN, -1).reshape(_NT, C, _K),
  ])
  o0 = dst < HN
  dst3d = jnp.stack([
      jnp.where(o0, dst, -1).reshape(E // _K, 1, _K),
      jnp.where(o0, -1, dst - HN).reshape(E // _K, 1, _K),
  ])

  q, ht, cum = _make_phase0(NP, D, E)(p_pad, relation_effect, dst3d)
  step = _make_iter(NP, D, C)
  for _ in range(_ITRS):
    ht, cum = step(srcp, dstp, q, ht, cum)
  return cum[:N]
